# baseline (speedup 1.0000x reference)
import functools

import jax
import jax.numpy as jnp
from jax import lax
from jax.experimental import pallas as pl
from jax.experimental.pallas import tpu as pltpu
from jax.experimental.pallas import tpu_sc as plsc

N = 10000
E = 160000
IN = 256
H = 512
OUT = 128
G = 64
EPS = 1e-5

NS = 16
NCHUNK = 4
FC = H // NCHUNK
EB = 128
NBLK = 80
EPAD = NS * NBLK * EB
NACC = 10112
RPT = NACC // NS
NB = 25
BN = N // NB


@functools.partial(
    pl.kernel,
    out_type=jax.ShapeDtypeStruct((NACC, 128), jnp.float32),
    mesh=plsc.VectorSubcoreMesh(core_axis_name="c", subcore_axis_name="s"),
    scratch_types=[
        pltpu.VMEM((NBLK, EB), jnp.int32),
        pltpu.VMEM((EB, 128), jnp.float32),
        pltpu.VMEM_SHARED((NACC, 128), jnp.float32),
    ],
)
def _deg_sc(dst_hbm, out_hbm, dst_v, buf, acc):
    c = lax.axis_index("c")
    s = lax.axis_index("s")
    base = s * RPT

    @pl.when(c == 0)
    def _():
        def fill_zero(i, _):
            for k in range(8):
                buf[i, pl.ds(k * 16, 16)] = jnp.zeros((16,), jnp.float32)
            return 0
        lax.fori_loop(0, EB, fill_zero, 0)
        for r in range(RPT // EB):
            pltpu.sync_copy(buf, acc.at[pl.ds(base + r * EB, EB)])
        pltpu.sync_copy(buf.at[pl.ds(0, RPT % EB)],
                        acc.at[pl.ds(base + (RPT // EB) * EB, RPT % EB)])

        def fill_ones(i, _):
            for k in range(8):
                buf[i, pl.ds(k * 16, 16)] = jnp.ones((16,), jnp.float32)
            return 0
        lax.fori_loop(0, EB, fill_ones, 0)
        pltpu.sync_copy(dst_hbm.at[s], dst_v)

    plsc.subcore_barrier()

    @pl.when(c == 0)
    def _():
        def body(j, _):
            pltpu.sync_copy(buf, acc.at[dst_v.at[j]], add=True)
            return 0
        lax.fori_loop(0, NBLK, body, 0)

    plsc.subcore_barrier()

    @pl.when(c == 0)
    def _():
        pltpu.sync_copy(acc.at[pl.ds(base, RPT)], out_hbm.at[pl.ds(base, RPT)])


GRP = 40


@functools.partial(
    pl.kernel,
    out_type=[jax.ShapeDtypeStruct((NACC, FC), jnp.float32)] * NCHUNK,
    mesh=plsc.VectorSubcoreMesh(core_axis_name="c", subcore_axis_name="s"),
    scratch_types=[
        pltpu.VMEM((GRP, EB), jnp.int32),
        pltpu.VMEM((GRP, EB), jnp.int32),
        pltpu.VMEM((EB, FC), jnp.float32),
        pltpu.VMEM((EB, FC), jnp.float32),
        pltpu.VMEM_SHARED((NACC, FC), jnp.float32),
        pltpu.SemaphoreType.DMA,
        pltpu.SemaphoreType.DMA,
        pltpu.SemaphoreType.DMA,
        pltpu.SemaphoreType.DMA,
    ],
)
def _agg_sc(z0, z1, z2, z3, src_hbm, dst_hbm, o0, o1, o2, o3,
            src_i, dst_i, gb0, gb1, acc, gsem0, gsem1, ssem0, ssem1):
    c = lax.axis_index("c")
    s = lax.axis_index("s")
    base = s * RPT

    def process(tin, tout):
        def gather(j, gb, gsem):
            pltpu.make_async_copy(tin.at[src_i.at[j]], gb, gsem).start()

        def fill_zero(i, _):
            for k in range(FC // 16):
                gb0[i, pl.ds(k * 16, 16)] = jnp.zeros((16,), jnp.float32)
            return 0
        lax.fori_loop(0, EB, fill_zero, 0)
        for r in range(RPT // EB):
            pltpu.sync_copy(gb0, acc.at[pl.ds(base + r * EB, EB)])
        pltpu.sync_copy(gb0.at[pl.ds(0, RPT % EB)],
                        acc.at[pl.ds(base + (RPT // EB) * EB, RPT % EB)])
        plsc.subcore_barrier()

        def group(g, _):
            pltpu.sync_copy(src_hbm.at[s, pl.ds(g * GRP, GRP)], src_i)
            pltpu.sync_copy(dst_hbm.at[s, pl.ds(g * GRP, GRP)], dst_i)
            gather(0, gb0, gsem0)
            gather(1, gb1, gsem1)

            def body(jj, _):
                j0 = 2 * jj
                pltpu.make_async_copy(tin.at[src_i.at[j0]], gb0, gsem0).wait()
                pltpu.make_async_copy(gb0, acc.at[dst_i.at[j0]],
                                      ssem0).start(add=True)
                pltpu.make_async_copy(tin.at[src_i.at[j0 + 1]], gb1,
                                      gsem1).wait()
                pltpu.make_async_copy(gb1, acc.at[dst_i.at[j0 + 1]],
                                      ssem1).start(add=True)

                @pl.when(j0 + 2 < GRP)
                def _():
                    pltpu.make_async_copy(gb0, acc.at[dst_i.at[j0]],
                                          ssem0).wait()
                    gather(j0 + 2, gb0, gsem0)

                @pl.when(j0 + 3 < GRP)
                def _():
                    pltpu.make_async_copy(gb1, acc.at[dst_i.at[j0 + 1]],
                                          ssem1).wait()
                    gather(j0 + 3, gb1, gsem1)
                return 0
            lax.fori_loop(0, GRP // 2, body, 0)
            pltpu.make_async_copy(gb0, acc.at[dst_i.at[GRP - 2]], ssem0).wait()
            pltpu.make_async_copy(gb1, acc.at[dst_i.at[GRP - 1]], ssem1).wait()
            return 0
        lax.fori_loop(0, NBLK // GRP, group, 0)
        plsc.subcore_barrier()
        pltpu.sync_copy(acc.at[pl.ds(base, RPT)], tout.at[pl.ds(base, RPT)])
        plsc.subcore_barrier()

    tables = (z0, z1, z2, z3)
    outs = (o0, o1, o2, o3)
    for chunk in range(NCHUNK):
        @pl.when(c == chunk // 2)
        def _(chunk=chunk):
            process(tables[chunk], outs[chunk])


def _dinv_of(deg_blk):
    return lax.rsqrt(deg_blk[:, 0:1] + 1.0)


def _k1_body(x_ref, w_ref, deg_ref, o0, o1, o2, o3):
    z = jnp.dot(x_ref[...], w_ref[...], preferred_element_type=jnp.float32)
    z = z * _dinv_of(deg_ref[...])
    o0[...] = z[:, 0 * FC:1 * FC]
    o1[...] = z[:, 1 * FC:2 * FC]
    o2[...] = z[:, 2 * FC:3 * FC]
    o3[...] = z[:, 3 * FC:4 * FC]


_k1_call = pl.pallas_call(
    _k1_body,
    grid=(NB,),
    in_specs=[
        pl.BlockSpec((BN, IN), lambda i: (i, 0)),
        pl.BlockSpec((IN, H), lambda i: (0, 0)),
        pl.BlockSpec((BN, 128), lambda i: (i, 0)),
    ],
    out_specs=[pl.BlockSpec((BN, FC), lambda i: (i, 0))] * NCHUNK,
    out_shape=[jax.ShapeDtypeStruct((N, FC), jnp.float32)] * NCHUNK,
)


def _k3_body(a0, a1, a2, a3, z0, z1, z2, z3, deg_ref, b_ref, h_ref, st_ref):
    i = pl.program_id(0)
    agg = jnp.concatenate([a0[...], a1[...], a2[...], a3[...]], axis=1)
    zc = jnp.concatenate([z0[...], z1[...], z2[...], z3[...]], axis=1)
    h = jnp.maximum((agg + zc) * _dinv_of(deg_ref[...]) + b_ref[...], 0.0)
    h_ref[...] = h

    @pl.when(i == 0)
    def _():
        st_ref[...] = jnp.zeros_like(st_ref)
    st_ref[0:1, :] += jnp.sum(h, axis=0, keepdims=True)
    st_ref[1:2, :] += jnp.sum(h * h, axis=0, keepdims=True)


_k3_call = pl.pallas_call(
    _k3_body,
    grid=(NB,),
    in_specs=[pl.BlockSpec((BN, FC), lambda i: (i, 0))] * NCHUNK
    + [pl.BlockSpec((BN, FC), lambda i: (i, 0))] * NCHUNK
    + [
        pl.BlockSpec((BN, 128), lambda i: (i, 0)),
        pl.BlockSpec((1, H), lambda i: (0, 0)),
    ],
    out_specs=[
        pl.BlockSpec((BN, H), lambda i: (i, 0)),
        pl.BlockSpec((8, H), lambda i: (0, 0)),
    ],
    out_shape=[
        jax.ShapeDtypeStruct((N, H), jnp.float32),
        jax.ShapeDtypeStruct((8, H), jnp.float32),
    ],
)


def _bn_affine(st, g, be):
    mean = st[0:1, :] * (1.0 / N)
    var = st[1:2, :] * (1.0 / N) - mean * mean
    a = g * lax.rsqrt(var + EPS)
    cvec = be - mean * a
    return a, cvec


def _k4_body(h_ref, st_ref, deg_ref, g_ref, be_ref, w_ref, o0, o1, o2, o3):
    a, cvec = _bn_affine(st_ref[...], g_ref[...], be_ref[...])
    t = jnp.dot(h_ref[...] * a, w_ref[...], preferred_element_type=jnp.float32)
    t = t + jnp.dot(cvec, w_ref[...], preferred_element_type=jnp.float32)
    t = t * _dinv_of(deg_ref[...])
    o0[...] = t[:, 0 * FC:1 * FC]
    o1[...] = t[:, 1 * FC:2 * FC]
    o2[...] = t[:, 2 * FC:3 * FC]
    o3[...] = t[:, 3 * FC:4 * FC]


_k4_call = pl.pallas_call(
    _k4_body,
    grid=(NB,),
    in_specs=[
        pl.BlockSpec((BN, H), lambda i: (i, 0)),
        pl.BlockSpec((8, H), lambda i: (0, 0)),
        pl.BlockSpec((BN, 128), lambda i: (i, 0)),
        pl.BlockSpec((1, H), lambda i: (0, 0)),
        pl.BlockSpec((1, H), lambda i: (0, 0)),
        pl.BlockSpec((H, H), lambda i: (0, 0)),
    ],
    out_specs=[pl.BlockSpec((BN, FC), lambda i: (i, 0))] * NCHUNK,
    out_shape=[jax.ShapeDtypeStruct((N, FC), jnp.float32)] * NCHUNK,
)


def _pool_body(h_ref, batch_ref, ps_ref, cnt_ref):
    i = pl.program_id(0)
    b = batch_ref[0, 0, :]
    m = (lax.broadcasted_iota(jnp.int32, (G, BN), 0) == b[None, :]).astype(
        jnp.float32)

    @pl.when(i == 0)
    def _():
        ps_ref[...] = jnp.zeros_like(ps_ref)
        cnt_ref[...] = jnp.zeros_like(cnt_ref)
    ps_ref[...] += jnp.dot(m, h_ref[...], preferred_element_type=jnp.float32)
    cnt_ref[...] += jnp.broadcast_to(
        jnp.sum(m, axis=1, keepdims=True), (G, 128))


_pool_call = pl.pallas_call(
    _pool_body,
    grid=(NB,),
    in_specs=[
        pl.BlockSpec((BN, H), lambda i: (i, 0)),
        pl.BlockSpec((1, 1, BN), lambda i: (i, 0, 0)),
    ],
    out_specs=[
        pl.BlockSpec((G, H), lambda i: (0, 0)),
        pl.BlockSpec((G, 128), lambda i: (0, 0)),
    ],
    out_shape=[
        jax.ShapeDtypeStruct((G, H), jnp.float32),
        jax.ShapeDtypeStruct((G, 128), jnp.float32),
    ],
)


def _head_body(ps_ref, cnt_ref, st_ref, g_ref, be_ref,
               fw1_ref, fb1_ref, fw2_ref, fb2_ref, out_ref):
    a, cvec = _bn_affine(st_ref[...], g_ref[...], be_ref[...])
    cnt = jnp.maximum(cnt_ref[:, 0:1], 1.0)
    pm = ps_ref[...] / cnt
    hb = pm * a + cvec
    r = jnp.maximum(
        jnp.dot(hb, fw1_ref[...], preferred_element_type=jnp.float32)
        + fb1_ref[...], 0.0)
    o = jnp.dot(r, fw2_ref[...], preferred_element_type=jnp.float32) + fb2_ref[...]
    mx = jnp.max(o, axis=1, keepdims=True)
    lse = jnp.log(jnp.sum(jnp.exp(o - mx), axis=1, keepdims=True)) + mx
    out_ref[...] = o - lse


_head_call = pl.pallas_call(
    _head_body,
    out_shape=jax.ShapeDtypeStruct((G, OUT), jnp.float32),
)


def kernel(x, edge_index, batch, W1, b1, W2, b2, g1, be1, g2, be2,
           fW1, fb1, fW2, fb2):
    src = edge_index[0].astype(jnp.int32)
    dst = edge_index[1].astype(jnp.int32)
    srcp = jnp.concatenate(
        [src, jnp.zeros((EPAD - E,), jnp.int32)]).reshape(NS, NBLK, EB)
    dstp = jnp.concatenate(
        [dst, jnp.full((EPAD - E,), N, jnp.int32)]).reshape(NS, NBLK, EB)
    batch3 = batch.astype(jnp.int32).reshape(NB, 1, BN)
    b1r = b1.reshape(1, H)
    b2r = b2.reshape(1, H)
    g1r = g1.reshape(1, H)
    be1r = be1.reshape(1, H)
    g2r = g2.reshape(1, H)
    be2r = be2.reshape(1, H)

    deg = _deg_sc(dstp)
    z1 = _k1_call(x, W1, deg)
    agg1 = _agg_sc(z1[0], z1[1], z1[2], z1[3], srcp, dstp)
    h1, st1 = _k3_call(agg1[0], agg1[1], agg1[2], agg1[3],
                       z1[0], z1[1], z1[2], z1[3], deg, b1r)
    z2 = _k4_call(h1, st1, deg, g1r, be1r, W2)
    agg2 = _agg_sc(z2[0], z2[1], z2[2], z2[3], srcp, dstp)
    h2, st2 = _k3_call(agg2[0], agg2[1], agg2[2], agg2[3],
                       z2[0], z2[1], z2[2], z2[3], deg, b2r)
    ps, cnt = _pool_call(h2, batch3)
    out = _head_call(ps, cnt, st2, g2r, be2r,
                     fW1, fb1.reshape(1, H // 2), fW2, fb2.reshape(1, OUT))
    return out


# speedup vs baseline: 1.1104x; 1.1104x over previous
import functools

import jax
import jax.numpy as jnp
from jax import lax
from jax.experimental import pallas as pl
from jax.experimental.pallas import tpu as pltpu
from jax.experimental.pallas import tpu_sc as plsc

N = 10000
E = 160000
IN = 256
H = 512
OUT = 128
G = 64
EPS = 1e-5

NS = 16
NCHUNK = 4
FC = H // NCHUNK
EB = 128
NBLK = 80
EPAD = NS * NBLK * EB
NACC = 10112
RPT = NACC // NS
NB = 25
BN = N // NB


@functools.partial(
    pl.kernel,
    out_type=[jax.ShapeDtypeStruct((NACC, 128), jnp.float32)] * 2,
    mesh=plsc.VectorSubcoreMesh(core_axis_name="c", subcore_axis_name="s"),
    scratch_types=[
        pltpu.VMEM((NBLK // 2, EB), jnp.int32),
        pltpu.VMEM((EB, 128), jnp.float32),
        pltpu.VMEM_SHARED((NACC, 128), jnp.float32),
    ],
)
def _deg_sc(dst_hbm, out0_hbm, out1_hbm, dst_v, buf, acc):
    c = lax.axis_index("c")
    s = lax.axis_index("s")
    base = s * RPT
    half = NBLK // 2

    def fill_zero(i, _):
        for k in range(8):
            buf[i, pl.ds(k * 16, 16)] = jnp.zeros((16,), jnp.float32)
        return 0
    lax.fori_loop(0, EB, fill_zero, 0)
    for r in range(RPT // EB):
        pltpu.sync_copy(buf, acc.at[pl.ds(base + r * EB, EB)])
    pltpu.sync_copy(buf.at[pl.ds(0, RPT % EB)],
                    acc.at[pl.ds(base + (RPT // EB) * EB, RPT % EB)])

    def fill_ones(i, _):
        for k in range(8):
            buf[i, pl.ds(k * 16, 16)] = jnp.ones((16,), jnp.float32)
        return 0
    lax.fori_loop(0, EB, fill_ones, 0)
    pltpu.sync_copy(dst_hbm.at[s, pl.ds(c * half, half)], dst_v)
    plsc.subcore_barrier()

    def body(j, _):
        pltpu.sync_copy(buf, acc.at[dst_v.at[j]], add=True)
        return 0
    lax.fori_loop(0, half, body, 0)
    plsc.subcore_barrier()

    @pl.when(c == 0)
    def _():
        pltpu.sync_copy(acc.at[pl.ds(base, RPT)], out0_hbm.at[pl.ds(base, RPT)])

    @pl.when(c == 1)
    def _():
        pltpu.sync_copy(acc.at[pl.ds(base, RPT)], out1_hbm.at[pl.ds(base, RPT)])


GRP = 40


@functools.partial(
    pl.kernel,
    out_type=[jax.ShapeDtypeStruct((NACC, FC), jnp.float32)] * NCHUNK,
    mesh=plsc.VectorSubcoreMesh(core_axis_name="c", subcore_axis_name="s"),
    scratch_types=[
        pltpu.VMEM((GRP, EB), jnp.int32),
        pltpu.VMEM((GRP, EB), jnp.int32),
        pltpu.VMEM((EB, FC), jnp.float32),
        pltpu.VMEM((EB, FC), jnp.float32),
        pltpu.VMEM_SHARED((NACC, FC), jnp.float32),
        pltpu.SemaphoreType.DMA,
        pltpu.SemaphoreType.DMA,
    ],
)
def _agg_sc(z0, z1, z2, z3, src_hbm, dst_hbm, o0, o1, o2, o3,
            src_i, dst_i, gb0, gb1, acc, gsem0, gsem1):
    c = lax.axis_index("c")
    s = lax.axis_index("s")
    base = s * RPT

    def process(tin, tout):
        def gather(j, gb, gsem):
            pltpu.make_async_copy(tin.at[src_i.at[j]], gb, gsem).start()

        def fill_zero(i, _):
            for k in range(FC // 16):
                gb0[i, pl.ds(k * 16, 16)] = jnp.zeros((16,), jnp.float32)
            return 0
        lax.fori_loop(0, EB, fill_zero, 0)
        for r in range(RPT // EB):
            pltpu.sync_copy(gb0, acc.at[pl.ds(base + r * EB, EB)])
        pltpu.sync_copy(gb0.at[pl.ds(0, RPT % EB)],
                        acc.at[pl.ds(base + (RPT // EB) * EB, RPT % EB)])
        plsc.subcore_barrier()

        def group(g, _):
            pltpu.sync_copy(src_hbm.at[s, pl.ds(g * GRP, GRP)], src_i)
            pltpu.sync_copy(dst_hbm.at[s, pl.ds(g * GRP, GRP)], dst_i)
            gather(0, gb0, gsem0)

            def body(jj, _):
                j0 = 2 * jj
                gather(j0 + 1, gb1, gsem1)
                pltpu.make_async_copy(tin.at[src_i.at[j0]], gb0, gsem0).wait()
                pltpu.sync_copy(gb0, acc.at[dst_i.at[j0]], add=True)

                @pl.when(j0 + 2 < GRP)
                def _():
                    gather(j0 + 2, gb0, gsem0)
                pltpu.make_async_copy(tin.at[src_i.at[j0 + 1]], gb1,
                                      gsem1).wait()
                pltpu.sync_copy(gb1, acc.at[dst_i.at[j0 + 1]], add=True)
                return 0
            lax.fori_loop(0, GRP // 2, body, 0)
            return 0
        lax.fori_loop(0, NBLK // GRP, group, 0)
        plsc.subcore_barrier()
        pltpu.sync_copy(acc.at[pl.ds(base, RPT)], tout.at[pl.ds(base, RPT)])
        plsc.subcore_barrier()

    tables = (z0, z1, z2, z3)
    outs = (o0, o1, o2, o3)
    for chunk in range(NCHUNK):
        @pl.when(c == chunk // 2)
        def _(chunk=chunk):
            process(tables[chunk], outs[chunk])


def _dinv_of(d0_blk, d1_blk):
    return lax.rsqrt(d0_blk[:, 0:1] + d1_blk[:, 0:1] + 1.0)


def _k1_body(x_ref, w_ref, d0_ref, d1_ref, o0, o1, o2, o3):
    z = jnp.dot(x_ref[...], w_ref[...], preferred_element_type=jnp.float32)
    z = z * _dinv_of(d0_ref[...], d1_ref[...])
    o0[...] = z[:, 0 * FC:1 * FC]
    o1[...] = z[:, 1 * FC:2 * FC]
    o2[...] = z[:, 2 * FC:3 * FC]
    o3[...] = z[:, 3 * FC:4 * FC]


_k1_call = pl.pallas_call(
    _k1_body,
    grid=(NB,),
    in_specs=[
        pl.BlockSpec((BN, IN), lambda i: (i, 0)),
        pl.BlockSpec((IN, H), lambda i: (0, 0)),
        pl.BlockSpec((BN, 128), lambda i: (i, 0)),
        pl.BlockSpec((BN, 128), lambda i: (i, 0)),
    ],
    out_specs=[pl.BlockSpec((BN, FC), lambda i: (i, 0))] * NCHUNK,
    out_shape=[jax.ShapeDtypeStruct((N, FC), jnp.float32)] * NCHUNK,
)


def _k3_body(a0, a1, a2, a3, z0, z1, z2, z3, d0_ref, d1_ref, b_ref,
             h_ref, st_ref):
    i = pl.program_id(0)
    agg = jnp.concatenate([a0[...], a1[...], a2[...], a3[...]], axis=1)
    zc = jnp.concatenate([z0[...], z1[...], z2[...], z3[...]], axis=1)
    dinv = _dinv_of(d0_ref[...], d1_ref[...])
    h = jnp.maximum((agg + zc) * dinv + b_ref[...], 0.0)
    h_ref[...] = h

    @pl.when(i == 0)
    def _():
        st_ref[...] = jnp.zeros_like(st_ref)
    st_ref[0:1, :] += jnp.sum(h, axis=0, keepdims=True)
    st_ref[1:2, :] += jnp.sum(h * h, axis=0, keepdims=True)


_k3_call = pl.pallas_call(
    _k3_body,
    grid=(NB,),
    in_specs=[pl.BlockSpec((BN, FC), lambda i: (i, 0))] * NCHUNK
    + [pl.BlockSpec((BN, FC), lambda i: (i, 0))] * NCHUNK
    + [
        pl.BlockSpec((BN, 128), lambda i: (i, 0)),
        pl.BlockSpec((BN, 128), lambda i: (i, 0)),
        pl.BlockSpec((1, H), lambda i: (0, 0)),
    ],
    out_specs=[
        pl.BlockSpec((BN, H), lambda i: (i, 0)),
        pl.BlockSpec((8, H), lambda i: (0, 0)),
    ],
    out_shape=[
        jax.ShapeDtypeStruct((N, H), jnp.float32),
        jax.ShapeDtypeStruct((8, H), jnp.float32),
    ],
)


def _k6_body(a0, a1, a2, a3, z0, z1, z2, z3, d0_ref, d1_ref, b_ref,
             batch_ref, st_ref, ps_ref, cnt_ref):
    i = pl.program_id(0)
    agg = jnp.concatenate([a0[...], a1[...], a2[...], a3[...]], axis=1)
    zc = jnp.concatenate([z0[...], z1[...], z2[...], z3[...]], axis=1)
    dinv = _dinv_of(d0_ref[...], d1_ref[...])
    h = jnp.maximum((agg + zc) * dinv + b_ref[...], 0.0)

    @pl.when(i == 0)
    def _():
        st_ref[...] = jnp.zeros_like(st_ref)
        ps_ref[...] = jnp.zeros_like(ps_ref)
        cnt_ref[...] = jnp.zeros_like(cnt_ref)
    st_ref[0:1, :] += jnp.sum(h, axis=0, keepdims=True)
    st_ref[1:2, :] += jnp.sum(h * h, axis=0, keepdims=True)
    b = batch_ref[0, 0, :]
    m = (lax.broadcasted_iota(jnp.int32, (G, BN), 0) == b[None, :]).astype(
        jnp.float32)
    ps_ref[...] += jnp.dot(m, h, preferred_element_type=jnp.float32)
    cnt_ref[...] += jnp.broadcast_to(
        jnp.sum(m, axis=1, keepdims=True), (G, 128))


_k6_call = pl.pallas_call(
    _k6_body,
    grid=(NB,),
    in_specs=[pl.BlockSpec((BN, FC), lambda i: (i, 0))] * NCHUNK
    + [pl.BlockSpec((BN, FC), lambda i: (i, 0))] * NCHUNK
    + [
        pl.BlockSpec((BN, 128), lambda i: (i, 0)),
        pl.BlockSpec((BN, 128), lambda i: (i, 0)),
        pl.BlockSpec((1, H), lambda i: (0, 0)),
        pl.BlockSpec((1, 1, BN), lambda i: (i, 0, 0)),
    ],
    out_specs=[
        pl.BlockSpec((8, H), lambda i: (0, 0)),
        pl.BlockSpec((G, H), lambda i: (0, 0)),
        pl.BlockSpec((G, 128), lambda i: (0, 0)),
    ],
    out_shape=[
        jax.ShapeDtypeStruct((8, H), jnp.float32),
        jax.ShapeDtypeStruct((G, H), jnp.float32),
        jax.ShapeDtypeStruct((G, 128), jnp.float32),
    ],
)


def _bn_affine(st, g, be):
    mean = st[0:1, :] * (1.0 / N)
    var = st[1:2, :] * (1.0 / N) - mean * mean
    a = g * lax.rsqrt(var + EPS)
    cvec = be - mean * a
    return a, cvec


def _k4_body(h_ref, st_ref, d0_ref, d1_ref, g_ref, be_ref, w_ref,
             o0, o1, o2, o3):
    a, cvec = _bn_affine(st_ref[...], g_ref[...], be_ref[...])
    t = jnp.dot(h_ref[...] * a, w_ref[...], preferred_element_type=jnp.float32)
    t = t + jnp.dot(cvec, w_ref[...], preferred_element_type=jnp.float32)
    t = t * _dinv_of(d0_ref[...], d1_ref[...])
    o0[...] = t[:, 0 * FC:1 * FC]
    o1[...] = t[:, 1 * FC:2 * FC]
    o2[...] = t[:, 2 * FC:3 * FC]
    o3[...] = t[:, 3 * FC:4 * FC]


_k4_call = pl.pallas_call(
    _k4_body,
    grid=(NB,),
    in_specs=[
        pl.BlockSpec((BN, H), lambda i: (i, 0)),
        pl.BlockSpec((8, H), lambda i: (0, 0)),
        pl.BlockSpec((BN, 128), lambda i: (i, 0)),
        pl.BlockSpec((BN, 128), lambda i: (i, 0)),
        pl.BlockSpec((1, H), lambda i: (0, 0)),
        pl.BlockSpec((1, H), lambda i: (0, 0)),
        pl.BlockSpec((H, H), lambda i: (0, 0)),
    ],
    out_specs=[pl.BlockSpec((BN, FC), lambda i: (i, 0))] * NCHUNK,
    out_shape=[jax.ShapeDtypeStruct((N, FC), jnp.float32)] * NCHUNK,
)


def _head_body(ps_ref, cnt_ref, st_ref, g_ref, be_ref,
               fw1_ref, fb1_ref, fw2_ref, fb2_ref, out_ref):
    a, cvec = _bn_affine(st_ref[...], g_ref[...], be_ref[...])
    cnt = jnp.maximum(cnt_ref[:, 0:1], 1.0)
    pm = ps_ref[...] / cnt
    hb = pm * a + cvec
    r = jnp.maximum(
        jnp.dot(hb, fw1_ref[...], preferred_element_type=jnp.float32)
        + fb1_ref[...], 0.0)
    o = jnp.dot(r, fw2_ref[...], preferred_element_type=jnp.float32) + fb2_ref[...]
    mx = jnp.max(o, axis=1, keepdims=True)
    lse = jnp.log(jnp.sum(jnp.exp(o - mx), axis=1, keepdims=True)) + mx
    out_ref[...] = o - lse


_head_call = pl.pallas_call(
    _head_body,
    out_shape=jax.ShapeDtypeStruct((G, OUT), jnp.float32),
)


def kernel(x, edge_index, batch, W1, b1, W2, b2, g1, be1, g2, be2,
           fW1, fb1, fW2, fb2):
    src = edge_index[0].astype(jnp.int32)
    dst = edge_index[1].astype(jnp.int32)
    srcp = jnp.concatenate(
        [src, jnp.zeros((EPAD - E,), jnp.int32)]).reshape(NS, NBLK, EB)
    dstp = jnp.concatenate(
        [dst, jnp.full((EPAD - E,), N, jnp.int32)]).reshape(NS, NBLK, EB)
    batch3 = batch.astype(jnp.int32).reshape(NB, 1, BN)
    b1r = b1.reshape(1, H)
    b2r = b2.reshape(1, H)
    g1r = g1.reshape(1, H)
    be1r = be1.reshape(1, H)
    g2r = g2.reshape(1, H)
    be2r = be2.reshape(1, H)

    d0, d1 = _deg_sc(dstp)
    z1 = _k1_call(x, W1, d0, d1)
    agg1 = _agg_sc(z1[0], z1[1], z1[2], z1[3], srcp, dstp)
    h1, st1 = _k3_call(agg1[0], agg1[1], agg1[2], agg1[3],
                       z1[0], z1[1], z1[2], z1[3], d0, d1, b1r)
    z2 = _k4_call(h1, st1, d0, d1, g1r, be1r, W2)
    agg2 = _agg_sc(z2[0], z2[1], z2[2], z2[3], srcp, dstp)
    st2, ps, cnt = _k6_call(agg2[0], agg2[1], agg2[2], agg2[3],
                            z2[0], z2[1], z2[2], z2[3], d0, d1, b2r, batch3)
    out = _head_call(ps, cnt, st2, g2r, be2r,
                     fW1, fb1.reshape(1, H // 2), fW2, fb2.reshape(1, OUT))
    return out


# speedup vs baseline: 1.1108x; 1.0003x over previous
import functools

import jax
import jax.numpy as jnp
from jax import lax
from jax.experimental import pallas as pl
from jax.experimental.pallas import tpu as pltpu
from jax.experimental.pallas import tpu_sc as plsc

N = 10000
E = 160000
IN = 256
H = 512
OUT = 128
G = 64
EPS = 1e-5

NS = 16
NCHUNK = 4
FC = H // NCHUNK
EB = 128
NBLK = 80
EPAD = NS * NBLK * EB
GRP = 40
NACC = 10112
RPT = NACC // NS
NB = 25
BN = N // NB


@functools.partial(
    pl.kernel,
    out_type=[jax.ShapeDtypeStruct((NACC, 128), jnp.float32)] * 2,
    mesh=plsc.VectorSubcoreMesh(core_axis_name="c", subcore_axis_name="s"),
    scratch_types=[
        pltpu.VMEM((NBLK // 2, EB), jnp.int32),
        pltpu.VMEM((EB, 128), jnp.float32),
        pltpu.VMEM_SHARED((NACC, 128), jnp.float32),
    ],
)
def _deg_sc(dst_hbm, out0_hbm, out1_hbm, dst_v, buf, acc):
    c = lax.axis_index("c")
    s = lax.axis_index("s")
    base = s * RPT
    half = NBLK // 2

    def fill_zero(i, _):
        for k in range(8):
            buf[i, pl.ds(k * 16, 16)] = jnp.zeros((16,), jnp.float32)
        return 0
    lax.fori_loop(0, EB, fill_zero, 0)
    for r in range(RPT // EB):
        pltpu.sync_copy(buf, acc.at[pl.ds(base + r * EB, EB)])
    pltpu.sync_copy(buf.at[pl.ds(0, RPT % EB)],
                    acc.at[pl.ds(base + (RPT // EB) * EB, RPT % EB)])

    def fill_ones(i, _):
        for k in range(8):
            buf[i, pl.ds(k * 16, 16)] = jnp.ones((16,), jnp.float32)
        return 0
    lax.fori_loop(0, EB, fill_ones, 0)
    pltpu.sync_copy(dst_hbm.at[s, pl.ds(c * half, half)], dst_v)
    plsc.subcore_barrier()

    def body(j, _):
        pltpu.sync_copy(buf, acc.at[dst_v.at[j]], add=True)
        return 0
    lax.fori_loop(0, half, body, 0)
    plsc.subcore_barrier()

    @pl.when(c == 0)
    def _():
        pltpu.sync_copy(acc.at[pl.ds(base, RPT)], out0_hbm.at[pl.ds(base, RPT)])

    @pl.when(c == 1)
    def _():
        pltpu.sync_copy(acc.at[pl.ds(base, RPT)], out1_hbm.at[pl.ds(base, RPT)])


@functools.partial(
    pl.kernel,
    out_type=[jax.ShapeDtypeStruct((NACC, FC), jnp.float32)] * NCHUNK,
    mesh=plsc.VectorSubcoreMesh(core_axis_name="c", subcore_axis_name="s"),
    scratch_types=[
        pltpu.VMEM((GRP, EB), jnp.int32),
        pltpu.VMEM((GRP, EB), jnp.int32),
        pltpu.VMEM((EB, FC), jnp.float32),
        pltpu.VMEM((EB, FC), jnp.float32),
        pltpu.VMEM_SHARED((NACC, FC), jnp.float32),
        pltpu.SemaphoreType.DMA,
        pltpu.SemaphoreType.DMA,
    ],
)
def _agg_sc(z0, z1, z2, z3, src_hbm, dst_hbm, o0, o1, o2, o3,
            src_i, dst_i, gb0, gb1, acc, gsem0, gsem1):
    c = lax.axis_index("c")
    s = lax.axis_index("s")
    base = s * RPT

    def process(tin, tout):
        def gather(j, gb, gsem):
            pltpu.make_async_copy(tin.at[src_i.at[j]], gb, gsem).start()

        def fill_zero(i, _):
            for k in range(FC // 16):
                gb0[i, pl.ds(k * 16, 16)] = jnp.zeros((16,), jnp.float32)
            return 0
        lax.fori_loop(0, EB, fill_zero, 0)
        for r in range(RPT // EB):
            pltpu.sync_copy(gb0, acc.at[pl.ds(base + r * EB, EB)])
        pltpu.sync_copy(gb0.at[pl.ds(0, RPT % EB)],
                        acc.at[pl.ds(base + (RPT // EB) * EB, RPT % EB)])
        plsc.subcore_barrier()

        def group(g, _):
            pltpu.sync_copy(src_hbm.at[s, pl.ds(g * GRP, GRP)], src_i)
            pltpu.sync_copy(dst_hbm.at[s, pl.ds(g * GRP, GRP)], dst_i)
            gather(0, gb0, gsem0)

            def body(jj, _):
                j0 = 2 * jj
                gather(j0 + 1, gb1, gsem1)
                pltpu.make_async_copy(tin.at[src_i.at[j0]], gb0, gsem0).wait()
                pltpu.sync_copy(gb0, acc.at[dst_i.at[j0]], add=True)

                @pl.when(j0 + 2 < GRP)
                def _():
                    gather(j0 + 2, gb0, gsem0)
                pltpu.make_async_copy(tin.at[src_i.at[j0 + 1]], gb1,
                                      gsem1).wait()
                pltpu.sync_copy(gb1, acc.at[dst_i.at[j0 + 1]], add=True)
                return 0
            lax.fori_loop(0, GRP // 2, body, 0)
            return 0
        lax.fori_loop(0, NBLK // GRP, group, 0)
        plsc.subcore_barrier()
        pltpu.sync_copy(acc.at[pl.ds(base, RPT)], tout.at[pl.ds(base, RPT)])
        plsc.subcore_barrier()

    tables = (z0, z1, z2, z3)
    outs = (o0, o1, o2, o3)
    for chunk in range(NCHUNK):
        @pl.when(c == chunk // 2)
        def _(chunk=chunk):
            process(tables[chunk], outs[chunk])


def _dinv_of(d0_blk, d1_blk):
    return lax.rsqrt(d0_blk[:, 0:1] + d1_blk[:, 0:1] + 1.0)


def _k1_body(x_ref, w_ref, d0_ref, d1_ref, o0, o1, o2, o3):
    z = jnp.dot(x_ref[...], w_ref[...], preferred_element_type=jnp.float32)
    z = z * _dinv_of(d0_ref[...], d1_ref[...])
    o0[...] = z[:, 0 * FC:1 * FC]
    o1[...] = z[:, 1 * FC:2 * FC]
    o2[...] = z[:, 2 * FC:3 * FC]
    o3[...] = z[:, 3 * FC:4 * FC]


_k1_call = pl.pallas_call(
    _k1_body,
    grid=(NB,),
    in_specs=[
        pl.BlockSpec((BN, IN), lambda i: (i, 0)),
        pl.BlockSpec((IN, H), lambda i: (0, 0)),
        pl.BlockSpec((BN, 128), lambda i: (i, 0)),
        pl.BlockSpec((BN, 128), lambda i: (i, 0)),
    ],
    out_specs=[pl.BlockSpec((BN, FC), lambda i: (i, 0))] * NCHUNK,
    out_shape=[jax.ShapeDtypeStruct((N, FC), jnp.float32)] * NCHUNK,
)


def _h_of(a0, a1, a2, a3, z0, z1, z2, z3, d0_ref, d1_ref, b_ref):
    agg = jnp.concatenate([a0[...], a1[...], a2[...], a3[...]], axis=1)
    zc = jnp.concatenate([z0[...], z1[...], z2[...], z3[...]], axis=1)
    dinv = _dinv_of(d0_ref[...], d1_ref[...])
    return jnp.maximum((agg + zc) * dinv + b_ref[...], 0.0)


def _k3_body(a0, a1, a2, a3, z0, z1, z2, z3, d0_ref, d1_ref, b_ref,
             h_ref, st_ref):
    i = pl.program_id(0)
    h = _h_of(a0, a1, a2, a3, z0, z1, z2, z3, d0_ref, d1_ref, b_ref)
    h_ref[...] = h

    @pl.when(i == 0)
    def _():
        st_ref[...] = jnp.zeros_like(st_ref)
    st_ref[0:1, :] += jnp.sum(h, axis=0, keepdims=True)
    st_ref[1:2, :] += jnp.sum(h * h, axis=0, keepdims=True)


_k3_call = pl.pallas_call(
    _k3_body,
    grid=(NB,),
    in_specs=[pl.BlockSpec((BN, FC), lambda i: (i, 0))] * (2 * NCHUNK)
    + [
        pl.BlockSpec((BN, 128), lambda i: (i, 0)),
        pl.BlockSpec((BN, 128), lambda i: (i, 0)),
        pl.BlockSpec((1, H), lambda i: (0, 0)),
    ],
    out_specs=[
        pl.BlockSpec((BN, H), lambda i: (i, 0)),
        pl.BlockSpec((8, H), lambda i: (0, 0)),
    ],
    out_shape=[
        jax.ShapeDtypeStruct((N, H), jnp.float32),
        jax.ShapeDtypeStruct((8, H), jnp.float32),
    ],
)


def _k6_body(a0, a1, a2, a3, z0, z1, z2, z3, d0_ref, d1_ref, b_ref,
             batch_ref, st_ref, ps_ref, cnt_ref):
    i = pl.program_id(0)
    h = _h_of(a0, a1, a2, a3, z0, z1, z2, z3, d0_ref, d1_ref, b_ref)

    @pl.when(i == 0)
    def _():
        st_ref[...] = jnp.zeros_like(st_ref)
        ps_ref[...] = jnp.zeros_like(ps_ref)
        cnt_ref[...] = jnp.zeros_like(cnt_ref)
    st_ref[0:1, :] += jnp.sum(h, axis=0, keepdims=True)
    st_ref[1:2, :] += jnp.sum(h * h, axis=0, keepdims=True)
    b = batch_ref[0, 0, :]
    m = (lax.broadcasted_iota(jnp.int32, (G, BN), 0) == b[None, :]).astype(
        jnp.float32)
    ps_ref[...] += jnp.dot(m, h, preferred_element_type=jnp.float32)
    cnt_ref[...] += jnp.broadcast_to(
        jnp.sum(m, axis=1, keepdims=True), (G, 128))


_k6_call = pl.pallas_call(
    _k6_body,
    grid=(NB,),
    in_specs=[pl.BlockSpec((BN, FC), lambda i: (i, 0))] * (2 * NCHUNK)
    + [
        pl.BlockSpec((BN, 128), lambda i: (i, 0)),
        pl.BlockSpec((BN, 128), lambda i: (i, 0)),
        pl.BlockSpec((1, H), lambda i: (0, 0)),
        pl.BlockSpec((1, 1, BN), lambda i: (i, 0, 0)),
    ],
    out_specs=[
        pl.BlockSpec((8, H), lambda i: (0, 0)),
        pl.BlockSpec((G, H), lambda i: (0, 0)),
        pl.BlockSpec((G, 128), lambda i: (0, 0)),
    ],
    out_shape=[
        jax.ShapeDtypeStruct((8, H), jnp.float32),
        jax.ShapeDtypeStruct((G, H), jnp.float32),
        jax.ShapeDtypeStruct((G, 128), jnp.float32),
    ],
)


def _bn_affine(st, g, be):
    mean = st[0:1, :] * (1.0 / N)
    var = st[1:2, :] * (1.0 / N) - mean * mean
    a = g * lax.rsqrt(var + EPS)
    cvec = be - mean * a
    return a, cvec


def _k4_body(h_ref, st_ref, d0_ref, d1_ref, g_ref, be_ref, w_ref,
             o0, o1, o2, o3):
    a, cvec = _bn_affine(st_ref[...], g_ref[...], be_ref[...])
    t = jnp.dot(h_ref[...] * a, w_ref[...], preferred_element_type=jnp.float32)
    t = t + jnp.dot(cvec, w_ref[...], preferred_element_type=jnp.float32)
    t = t * _dinv_of(d0_ref[...], d1_ref[...])
    o0[...] = t[:, 0 * FC:1 * FC]
    o1[...] = t[:, 1 * FC:2 * FC]
    o2[...] = t[:, 2 * FC:3 * FC]
    o3[...] = t[:, 3 * FC:4 * FC]


_k4_call = pl.pallas_call(
    _k4_body,
    grid=(NB,),
    in_specs=[
        pl.BlockSpec((BN, H), lambda i: (i, 0)),
        pl.BlockSpec((8, H), lambda i: (0, 0)),
        pl.BlockSpec((BN, 128), lambda i: (i, 0)),
        pl.BlockSpec((BN, 128), lambda i: (i, 0)),
        pl.BlockSpec((1, H), lambda i: (0, 0)),
        pl.BlockSpec((1, H), lambda i: (0, 0)),
        pl.BlockSpec((H, H), lambda i: (0, 0)),
    ],
    out_specs=[pl.BlockSpec((BN, FC), lambda i: (i, 0))] * NCHUNK,
    out_shape=[jax.ShapeDtypeStruct((N, FC), jnp.float32)] * NCHUNK,
)


def _head_body(ps_ref, cnt_ref, st_ref, g_ref, be_ref,
               fw1_ref, fb1_ref, fw2_ref, fb2_ref, out_ref):
    a, cvec = _bn_affine(st_ref[...], g_ref[...], be_ref[...])
    cnt = jnp.maximum(cnt_ref[:, 0:1], 1.0)
    pm = ps_ref[...] / cnt
    hb = pm * a + cvec
    r = jnp.maximum(
        jnp.dot(hb, fw1_ref[...], preferred_element_type=jnp.float32)
        + fb1_ref[...], 0.0)
    o = jnp.dot(r, fw2_ref[...], preferred_element_type=jnp.float32) + fb2_ref[...]
    mx = jnp.max(o, axis=1, keepdims=True)
    lse = jnp.log(jnp.sum(jnp.exp(o - mx), axis=1, keepdims=True)) + mx
    out_ref[...] = o - lse


_head_call = pl.pallas_call(
    _head_body,
    out_shape=jax.ShapeDtypeStruct((G, OUT), jnp.float32),
)


def kernel(x, edge_index, batch, W1, b1, W2, b2, g1, be1, g2, be2,
           fW1, fb1, fW2, fb2):
    src = edge_index[0].astype(jnp.int32)
    dst = edge_index[1].astype(jnp.int32)
    srcp = jnp.concatenate(
        [src, jnp.zeros((EPAD - E,), jnp.int32)]).reshape(NS, NBLK, EB)
    dstp = jnp.concatenate(
        [dst, jnp.full((EPAD - E,), N, jnp.int32)]).reshape(NS, NBLK, EB)
    batch3 = batch.astype(jnp.int32).reshape(NB, 1, BN)
    b1r = b1.reshape(1, H)
    b2r = b2.reshape(1, H)
    g1r = g1.reshape(1, H)
    be1r = be1.reshape(1, H)
    g2r = g2.reshape(1, H)
    be2r = be2.reshape(1, H)

    d0, d1 = _deg_sc(dstp)
    z1 = _k1_call(x, W1, d0, d1)
    agg1 = _agg_sc(z1[0], z1[1], z1[2], z1[3], srcp, dstp)
    h1, st1 = _k3_call(agg1[0], agg1[1], agg1[2], agg1[3],
                       z1[0], z1[1], z1[2], z1[3], d0, d1, b1r)
    z2 = _k4_call(h1, st1, d0, d1, g1r, be1r, W2)
    agg2 = _agg_sc(z2[0], z2[1], z2[2], z2[3], srcp, dstp)
    st2, ps, cnt = _k6_call(agg2[0], agg2[1], agg2[2], agg2[3],
                            z2[0], z2[1], z2[2], z2[3], d0, d1, b2r, batch3)
    out = _head_call(ps, cnt, st2, g2r, be2r,
                     fW1, fb1.reshape(1, H // 2), fW2, fb2.reshape(1, OUT))
    return out
